# baseline (speedup 1.0000x reference)
import functools

import numpy as np

import jax
import jax.numpy as jnp
from jax import lax
from jax.experimental import pallas as pl
from jax.experimental.pallas import tpu as pltpu
from jax.experimental.pallas import tpu_sc as plsc

_N = 10000
_E = 320000
_D = 128
_Q = 128
_H = _Q // 2

_NC = 2
_NS = 16
_NW = _NC * _NS
_K = 80
_CHUNKS = 125
_EPW = _K * _CHUNKS
_NPAD = 10240
_RPT = _NPAD // _NS

_MM_BLK = 1000

def _pack_bf16_pairs(h):
    lo = lax.bitcast_convert_type(
        h[:, :_H].astype(jnp.bfloat16), jnp.uint16).astype(jnp.uint32)
    hi = lax.bitcast_convert_type(
        h[:, _H:].astype(jnp.bfloat16), jnp.uint16).astype(jnp.uint32)
    return lax.bitcast_convert_type(lo | (hi << 16), jnp.int32)


def _mm_body(merger_ref, x_ref, w0_ref, w1_ref, wi_ref, g0_ref, g1_ref, hi_ref):
    x = x_ref[...]
    g0_ref[...] = _pack_bf16_pairs(merger_ref[0, 0] * jnp.dot(
        x, w0_ref[...], preferred_element_type=jnp.float32))
    g1_ref[...] = _pack_bf16_pairs(merger_ref[0, 1] * jnp.dot(
        x, w1_ref[...], preferred_element_type=jnp.float32))
    hi_ref[...] = merger_ref[0, 2] * jnp.dot(x, wi_ref[...],
                                             preferred_element_type=jnp.float32)


def _projections(x, w0, w1, wi, merger):
    grid = _N // _MM_BLK
    row_spec = pl.BlockSpec((_MM_BLK, _D), lambda i: (i, 0))
    w_spec = pl.BlockSpec((_D, _Q), lambda i: (0, 0))
    out = pl.pallas_call(
        _mm_body,
        grid=(grid,),
        in_specs=[
            pl.BlockSpec(memory_space=pltpu.SMEM),
            row_spec, w_spec, w_spec, w_spec,
        ],
        out_specs=[
            pl.BlockSpec((_MM_BLK, _H), lambda i: (i, 0)),
            pl.BlockSpec((_MM_BLK, _H), lambda i: (i, 0)),
            pl.BlockSpec((_MM_BLK, _Q), lambda i: (i, 0)),
        ],
        out_shape=[
            jax.ShapeDtypeStruct((_N, _H), jnp.int32),
            jax.ShapeDtypeStruct((_N, _H), jnp.int32),
            jax.ShapeDtypeStruct((_N, _Q), jnp.float32),
        ],
        compiler_params=pltpu.CompilerParams(
            dimension_semantics=("parallel",)),
    )(merger.reshape(1, 3), x, w0, w1, wi)
    return out


def _sc_body(g0_hbm, g1_hbm, src_hbm, dst_hbm, ew_hbm, z_hbm, out_hbm,
             sv, dv, wv, grows_a, grows_b, frows_a, frows_b, acc,
             sg_a, sg_b, sx_a, sx_b, si):
    cid = lax.axis_index("c")
    sid = lax.axis_index("s")
    wid = sid * _NC + cid

    pltpu.sync_copy(z_hbm.at[pl.ds(sid * _RPT, _RPT)],
                    acc.at[pl.ds(sid * _RPT, _RPT)])
    plsc.subcore_barrier()

    base0 = wid * _EPW

    def idx_start(c):
        base = base0 + c * _K
        b = lax.rem(c, 3)
        pltpu.make_async_copy(src_hbm.at[pl.ds(base, _K)], sv.at[b], si).start()
        pltpu.make_async_copy(dst_hbm.at[pl.ds(base, _K)], dv.at[b], si).start()
        pltpu.make_async_copy(ew_hbm.at[pl.ds(base, _K)], wv.at[b], si).start()

    def idx_wait(c):
        base = base0 + c * _K
        b = lax.rem(c, 3)
        pltpu.make_async_copy(src_hbm.at[pl.ds(base, _K)], sv.at[b], si).wait()
        pltpu.make_async_copy(dst_hbm.at[pl.ds(base, _K)], dv.at[b], si).wait()
        pltpu.make_async_copy(ew_hbm.at[pl.ds(base, _K)], wv.at[b], si).wait()

    _KH = _K // 2

    def gathers_start(c):
        b = lax.rem(c, 3)
        p = lax.rem(c, 2)
        pltpu.make_async_copy(g0_hbm.at[sv.at[b, pl.ds(0, _KH)]],
                              grows_a.at[p, pl.ds(0, _KH)], sg_a).start()
        pltpu.make_async_copy(g1_hbm.at[dv.at[b, pl.ds(0, _KH)]],
                              grows_b.at[p, pl.ds(0, _KH)], sg_b).start()
        pltpu.make_async_copy(g0_hbm.at[sv.at[b, pl.ds(_KH, _KH)]],
                              grows_a.at[p, pl.ds(_KH, _KH)], sg_a).start()
        pltpu.make_async_copy(g1_hbm.at[dv.at[b, pl.ds(_KH, _KH)]],
                              grows_b.at[p, pl.ds(_KH, _KH)], sg_b).start()

    def scale(grows, frows, p, c):
        b = lax.rem(c, 3)
        mask_hi = jnp.full((16,), -65536, jnp.int32)

        def scale16(j16, c2):
            w16 = wv[b, pl.ds(j16 * 16, 16)]
            for l in range(16):
                j = j16 * 16 + l
                wb = lax.gather(
                    w16, jnp.full((16, 1), l, jnp.int32),
                    lax.GatherDimensionNumbers(
                        offset_dims=(), collapsed_slice_dims=(0,),
                        start_index_map=(0,)),
                    slice_sizes=(1,),
                    mode=lax.GatherScatterMode.PROMISE_IN_BOUNDS)
                packed = [grows[p, j, pl.ds(t * 16, 16)]
                          for t in range(_H // 16)]
                for t in range(_H // 16):
                    lo = plsc.bitcast(
                        lax.shift_left(packed[t], 16), jnp.float32)
                    hi = plsc.bitcast(
                        jnp.bitwise_and(packed[t], mask_hi), jnp.float32)
                    frows[j, pl.ds(t * 16, 16)] = lo * wb
                    frows[j, pl.ds(_H + t * 16, 16)] = hi * wb
            return c2

        lax.fori_loop(0, _K // 16, scale16, 0)

    def scatter_wait(c):
        b = lax.rem(c, 3)
        pltpu.make_async_copy(frows_a, acc.at[dv.at[b]], sx_a).wait()
        pltpu.make_async_copy(frows_b, acc.at[sv.at[b]], sx_b).wait()

    idx_start(0)
    idx_wait(0)
    gathers_start(0)
    idx_start(1)

    def chunk_body(c, carry):
        b = lax.rem(c, 3)
        p = lax.rem(c, 2)

        @pl.when(c + 1 < _CHUNKS)
        def _():
            idx_wait(c + 1)

        @pl.when(c >= 1)
        def _():
            scatter_wait(c - 1)

        @pl.when(c + 1 < _CHUNKS)
        def _():
            gathers_start(c + 1)

        @pl.when(c + 2 < _CHUNKS)
        def _():
            idx_start(c + 2)

        pltpu.make_async_copy(g0_hbm.at[sv.at[b, pl.ds(0, _KH)]],
                              grows_a.at[p, pl.ds(0, _KH)], sg_a).wait()
        pltpu.make_async_copy(g0_hbm.at[sv.at[b, pl.ds(_KH, _KH)]],
                              grows_a.at[p, pl.ds(_KH, _KH)], sg_a).wait()
        scale(grows_a, frows_a, p, c)
        pltpu.async_copy(frows_a, acc.at[dv.at[b]], sx_a, add=True)

        pltpu.make_async_copy(g1_hbm.at[dv.at[b, pl.ds(0, _KH)]],
                              grows_b.at[p, pl.ds(0, _KH)], sg_b).wait()
        pltpu.make_async_copy(g1_hbm.at[dv.at[b, pl.ds(_KH, _KH)]],
                              grows_b.at[p, pl.ds(_KH, _KH)], sg_b).wait()
        scale(grows_b, frows_b, p, c)
        pltpu.async_copy(frows_b, acc.at[sv.at[b]], sx_b, add=True)
        return carry

    lax.fori_loop(0, _CHUNKS, chunk_body, 0)
    scatter_wait(_CHUNKS - 1)

    plsc.subcore_barrier()
    pltpu.sync_copy(acc.at[pl.ds(sid * _RPT, _RPT)],
                    out_hbm.at[cid, pl.ds(sid * _RPT, _RPT)])


_sc_edges = functools.partial(
    pl.kernel,
    out_type=jax.ShapeDtypeStruct((_NC, _NPAD, _Q), jnp.float32),
    mesh=plsc.VectorSubcoreMesh(core_axis_name="c", subcore_axis_name="s"),
    compiler_params=pltpu.CompilerParams(needs_layout_passes=False,
                                         use_tc_tiling_on_sc=False),
    scratch_types=[
        pltpu.VMEM((3, _K), jnp.int32),
        pltpu.VMEM((3, _K), jnp.int32),
        pltpu.VMEM((3, _K), jnp.float32),
        pltpu.VMEM((2, _K, _H), jnp.int32),
        pltpu.VMEM((2, _K, _H), jnp.int32),
        pltpu.VMEM((_K, _Q), jnp.float32),
        pltpu.VMEM((_K, _Q), jnp.float32),
        pltpu.VMEM_SHARED((_NPAD, _Q), jnp.float32),
        pltpu.SemaphoreType.DMA,
        pltpu.SemaphoreType.DMA,
        pltpu.SemaphoreType.DMA,
        pltpu.SemaphoreType.DMA,
        pltpu.SemaphoreType.DMA,
    ],
)(_sc_body)


def kernel(macro_features, edge_index, edge_weight, W0, W1, W_inner, merger):
    src = edge_index[0].astype(jnp.int32)
    dst = edge_index[1].astype(jnp.int32)
    zeros = jnp.zeros((_NPAD, _Q), jnp.float32)

    g0, g1, hi = _projections(macro_features, W0, W1, W_inner, merger)
    parts = _sc_edges(g0, g1, src, dst, edge_weight, zeros)
    return hi + parts[0, :_N] + parts[1, :_N]


# speedup vs baseline: 1.0132x; 1.0132x over previous
import functools

import numpy as np

import jax
import jax.numpy as jnp
from jax import lax
from jax.experimental import pallas as pl
from jax.experimental.pallas import tpu as pltpu
from jax.experimental.pallas import tpu_sc as plsc

_N = 10000
_E = 320000
_D = 128
_Q = 128
_H = _Q // 2

_NC = 2
_NS = 16
_NW = _NC * _NS
_K = 80
_CHUNKS = 125
_EPW = _K * _CHUNKS
_NPAD = 10240
_RPT = _NPAD // _NS

_MM_BLK = 1000

def _pack_bf16_pairs(h):
    lo = lax.bitcast_convert_type(
        h[:, :_H].astype(jnp.bfloat16), jnp.uint16).astype(jnp.uint32)
    hi = lax.bitcast_convert_type(
        h[:, _H:].astype(jnp.bfloat16), jnp.uint16).astype(jnp.uint32)
    return lax.bitcast_convert_type(lo | (hi << 16), jnp.int32)


def _mm_body(merger_ref, x_ref, w0_ref, w1_ref, wi_ref, g0_ref, g1_ref, hi_ref):
    x = x_ref[...]
    g0_ref[...] = _pack_bf16_pairs(merger_ref[0, 0] * jnp.dot(
        x, w0_ref[...], preferred_element_type=jnp.float32))
    g1_ref[...] = _pack_bf16_pairs(merger_ref[0, 1] * jnp.dot(
        x, w1_ref[...], preferred_element_type=jnp.float32))
    hi_ref[...] = (0.5 * merger_ref[0, 2]) * jnp.dot(
        x, wi_ref[...], preferred_element_type=jnp.float32)


def _projections(x, w0, w1, wi, merger):
    grid = _N // _MM_BLK
    row_spec = pl.BlockSpec((_MM_BLK, _D), lambda i: (i, 0))
    w_spec = pl.BlockSpec((_D, _Q), lambda i: (0, 0))
    out = pl.pallas_call(
        _mm_body,
        grid=(grid,),
        in_specs=[
            pl.BlockSpec(memory_space=pltpu.SMEM),
            row_spec, w_spec, w_spec, w_spec,
        ],
        out_specs=[
            pl.BlockSpec((_MM_BLK, _H), lambda i: (i, 0)),
            pl.BlockSpec((_MM_BLK, _H), lambda i: (i, 0)),
            pl.BlockSpec((_MM_BLK, _Q), lambda i: (i, 0)),
        ],
        out_shape=[
            jax.ShapeDtypeStruct((_N, _H), jnp.int32),
            jax.ShapeDtypeStruct((_N, _H), jnp.int32),
            jax.ShapeDtypeStruct((_N, _Q), jnp.float32),
        ],
        compiler_params=pltpu.CompilerParams(
            dimension_semantics=("parallel",)),
    )(merger.reshape(1, 3), x, w0, w1, wi)
    return out


def _sc_body(g0_hbm, g1_hbm, src_hbm, dst_hbm, ew_hbm, hh_hbm, out_hbm,
             sv, dv, wv, grows_a, grows_b, frows_a, frows_b, acc,
             sg_a, sg_b, sx_a, sx_b, si):
    cid = lax.axis_index("c")
    sid = lax.axis_index("s")
    wid = sid * _NC + cid

    @pl.when(sid < _NS - 1)
    def _():
        pltpu.sync_copy(hh_hbm.at[pl.ds(sid * _RPT, _RPT)],
                        acc.at[pl.ds(sid * _RPT, _RPT)])

    @pl.when(sid == _NS - 1)
    def _():
        pltpu.sync_copy(hh_hbm.at[pl.ds((_NS - 1) * _RPT, _N - (_NS - 1) * _RPT)],
                        acc.at[pl.ds((_NS - 1) * _RPT, _N - (_NS - 1) * _RPT)])

    plsc.subcore_barrier()

    base0 = wid * _EPW

    def idx_start(c):
        base = base0 + c * _K
        b = lax.rem(c, 3)
        pltpu.make_async_copy(src_hbm.at[pl.ds(base, _K)], sv.at[b], si).start()
        pltpu.make_async_copy(dst_hbm.at[pl.ds(base, _K)], dv.at[b], si).start()
        pltpu.make_async_copy(ew_hbm.at[pl.ds(base, _K)], wv.at[b], si).start()

    def idx_wait(c):
        base = base0 + c * _K
        b = lax.rem(c, 3)
        pltpu.make_async_copy(src_hbm.at[pl.ds(base, _K)], sv.at[b], si).wait()
        pltpu.make_async_copy(dst_hbm.at[pl.ds(base, _K)], dv.at[b], si).wait()
        pltpu.make_async_copy(ew_hbm.at[pl.ds(base, _K)], wv.at[b], si).wait()

    def gathers_start(c):
        b = lax.rem(c, 3)
        p = lax.rem(c, 2)
        pltpu.make_async_copy(g0_hbm.at[sv.at[b]], grows_a.at[p], sg_a).start()
        pltpu.make_async_copy(g1_hbm.at[dv.at[b]], grows_b.at[p], sg_b).start()

    def scale(grows, frows, p, c):
        b = lax.rem(c, 3)
        mask_hi = jnp.full((16,), -65536, jnp.int32)

        def scale16(j16, c2):
            w16 = wv[b, pl.ds(j16 * 16, 16)]
            for l in range(16):
                j = j16 * 16 + l
                wb = lax.gather(
                    w16, jnp.full((16, 1), l, jnp.int32),
                    lax.GatherDimensionNumbers(
                        offset_dims=(), collapsed_slice_dims=(0,),
                        start_index_map=(0,)),
                    slice_sizes=(1,),
                    mode=lax.GatherScatterMode.PROMISE_IN_BOUNDS)
                packed = [grows[p, j, pl.ds(t * 16, 16)]
                          for t in range(_H // 16)]
                for t in range(_H // 16):
                    lo = plsc.bitcast(
                        lax.shift_left(packed[t], 16), jnp.float32)
                    hi = plsc.bitcast(
                        jnp.bitwise_and(packed[t], mask_hi), jnp.float32)
                    frows[j, pl.ds(t * 16, 16)] = lo * wb
                    frows[j, pl.ds(_H + t * 16, 16)] = hi * wb
            return c2

        lax.fori_loop(0, _K // 16, scale16, 0)

    def scatter_wait(c):
        b = lax.rem(c, 3)
        pltpu.make_async_copy(frows_a, acc.at[dv.at[b]], sx_a).wait()
        pltpu.make_async_copy(frows_b, acc.at[sv.at[b]], sx_b).wait()

    idx_start(0)
    idx_wait(0)
    gathers_start(0)
    idx_start(1)

    def chunk_body(c, carry):
        b = lax.rem(c, 3)
        p = lax.rem(c, 2)

        @pl.when(c + 1 < _CHUNKS)
        def _():
            idx_wait(c + 1)

        @pl.when(c >= 1)
        def _():
            scatter_wait(c - 1)

        @pl.when(c + 1 < _CHUNKS)
        def _():
            gathers_start(c + 1)

        @pl.when(c + 2 < _CHUNKS)
        def _():
            idx_start(c + 2)

        pltpu.make_async_copy(g0_hbm.at[sv.at[b]], grows_a.at[p], sg_a).wait()
        scale(grows_a, frows_a, p, c)
        pltpu.async_copy(frows_a, acc.at[dv.at[b]], sx_a, add=True)

        pltpu.make_async_copy(g1_hbm.at[dv.at[b]], grows_b.at[p], sg_b).wait()
        scale(grows_b, frows_b, p, c)
        pltpu.async_copy(frows_b, acc.at[sv.at[b]], sx_b, add=True)
        return carry

    lax.fori_loop(0, _CHUNKS, chunk_body, 0)
    scatter_wait(_CHUNKS - 1)

    plsc.subcore_barrier()
    pltpu.sync_copy(acc.at[pl.ds(sid * _RPT, _RPT)],
                    out_hbm.at[cid, pl.ds(sid * _RPT, _RPT)])


_sc_edges = functools.partial(
    pl.kernel,
    out_type=jax.ShapeDtypeStruct((_NC, _NPAD, _Q), jnp.float32),
    mesh=plsc.VectorSubcoreMesh(core_axis_name="c", subcore_axis_name="s"),
    compiler_params=pltpu.CompilerParams(needs_layout_passes=False,
                                         use_tc_tiling_on_sc=False),
    scratch_types=[
        pltpu.VMEM((3, _K), jnp.int32),
        pltpu.VMEM((3, _K), jnp.int32),
        pltpu.VMEM((3, _K), jnp.float32),
        pltpu.VMEM((2, _K, _H), jnp.int32),
        pltpu.VMEM((2, _K, _H), jnp.int32),
        pltpu.VMEM((_K, _Q), jnp.float32),
        pltpu.VMEM((_K, _Q), jnp.float32),
        pltpu.VMEM_SHARED((_NPAD, _Q), jnp.float32),
        pltpu.SemaphoreType.DMA,
        pltpu.SemaphoreType.DMA,
        pltpu.SemaphoreType.DMA,
        pltpu.SemaphoreType.DMA,
        pltpu.SemaphoreType.DMA,
    ],
)(_sc_body)


def kernel(macro_features, edge_index, edge_weight, W0, W1, W_inner, merger):
    src = edge_index[0].astype(jnp.int32)
    dst = edge_index[1].astype(jnp.int32)

    g0, g1, hh = _projections(macro_features, W0, W1, W_inner, merger)
    parts = _sc_edges(g0, g1, src, dst, edge_weight, hh)
    return parts[0, :_N] + parts[1, :_N]
